# baseline (speedup 1.0000x reference)
import functools

import jax
import jax.numpy as jnp
import numpy as np
from jax import lax
from jax.experimental import pallas as pl
from jax.experimental.pallas import tpu as pltpu
from jax.experimental.pallas import tpu_sc as plsc

_MAX_PLAYER_NUMBER = 9
_LEN_CARD_FIGURE = 13
_LEN_CARD_DECOR = 4
_LEN_PLAYER_STATUS = 4
_NUM_BINS = 100
_NUM_STARTERS = 1
_EMBED_DIM = 128
_BATCH = 4096


def _field_offsets():
    field_dim_list = [
        (1, 4),
        (1, _MAX_PLAYER_NUMBER + 1),
        (7 * 2, _LEN_CARD_FIGURE + 1),
        (7 * 2, _LEN_CARD_DECOR + 1),
        (1, _NUM_BINS),
        (1, _NUM_BINS),
        (1, _NUM_BINS),
        (1, _NUM_BINS),
        (1, _NUM_BINS + 1),
        (1, _NUM_BINS + 1),
        (1, _NUM_BINS),
        (1, _NUM_BINS + 1),
        (1, _NUM_BINS + 1),
        (_MAX_PLAYER_NUMBER - 1, _LEN_PLAYER_STATUS),
        (_MAX_PLAYER_NUMBER - 1, _NUM_BINS),
        (_MAX_PLAYER_NUMBER - 1, _NUM_BINS + 1),
    ]
    cur = _NUM_STARTERS
    lst = []
    for num_fields, num_dims in field_dim_list:
        lst.extend([cur] * num_fields)
        cur += num_dims
    return np.asarray(lst, dtype=np.int32)

_STARTS = _field_offsets()
_NUM_FIELDS = _STARTS.shape[0]
_FIELDS = _NUM_FIELDS + _NUM_STARTERS

_NC, _NS = 2, 16
_NW = _NC * _NS
_ROWS = _BATCH * _FIELDS
_RPW = _ROWS // _NW
_CHUNK = 256
_GSZ = 128
_NG = _CHUNK // _GSZ
_NCHUNK = _RPW // _CHUNK

_OFF64 = np.concatenate([np.zeros((1,), np.int32), _STARTS])
_OFF_TILE = np.tile(_OFF64, _CHUNK // _FIELDS)


def _sc_gather(xflat, table, off_tile):
    mesh = plsc.VectorSubcoreMesh(
        core_axis_name="c", subcore_axis_name="s",
        num_cores=_NC, num_subcores=_NS)

    @functools.partial(
        pl.kernel,
        out_type=jax.ShapeDtypeStruct((_ROWS, _EMBED_DIM), jnp.float32),
        mesh=mesh,
        scratch_types=[
            pltpu.VMEM((2, _CHUNK), jnp.int32),
            pltpu.VMEM((_CHUNK,), jnp.int32),
            [pltpu.VMEM((2, _GSZ), jnp.int32) for _ in range(_NG)],
            pltpu.VMEM((2, _CHUNK, _EMBED_DIM), jnp.float32),
            pltpu.SemaphoreType.DMA,
            pltpu.SemaphoreType.DMA,
            pltpu.SemaphoreType.DMA,
        ],
    )
    def k(x_hbm, tab_hbm, off_hbm, out_hbm, x_v, off_v, idx_v, rows_v,
          ld_sem, g_sem, st_sem):
        wid = lax.axis_index("s") * _NC + lax.axis_index("c")
        base = wid * _RPW
        pltpu.sync_copy(off_hbm, off_v)

        def load(t, slot):
            pltpu.async_copy(
                x_hbm.at[pl.ds(base + t * _CHUNK, _CHUNK)],
                x_v.at[slot], ld_sem)

        def gather(slot):
            for j in range(_NG):
                for kk in range(_GSZ // 16):
                    s = pl.ds(kk * 16, 16)
                    cs = pl.ds(j * _GSZ + kk * 16, 16)
                    idx_v[j][slot, s] = x_v[slot, cs] + off_v[cs]
            for j in range(_NG):
                pltpu.async_copy(
                    tab_hbm.at[idx_v[j].at[slot]],
                    rows_v.at[slot, pl.ds(j * _GSZ, _GSZ)], g_sem)

        def store(t, slot):
            pltpu.async_copy(
                rows_v.at[slot],
                out_hbm.at[pl.ds(base + t * _CHUNK, _CHUNK)], st_sem)

        def wait_ld(slot):
            pltpu.make_async_copy(
                x_hbm.at[pl.ds(base, _CHUNK)], x_v.at[slot], ld_sem).wait()

        def wait_g(slot):
            for j in range(_NG):
                pltpu.make_async_copy(
                    tab_hbm.at[pl.ds(0, _GSZ)],
                    rows_v.at[slot, pl.ds(j * _GSZ, _GSZ)], g_sem).wait()

        def wait_st(slot):
            pltpu.make_async_copy(
                rows_v.at[slot],
                out_hbm.at[pl.ds(base, _CHUNK)], st_sem).wait()

        load(0, 0)
        wait_ld(0)
        gather(0)
        for t in range(_NCHUNK):
            slot = t % 2
            nxt = 1 - slot
            if t + 1 < _NCHUNK:
                load(t + 1, nxt)
            wait_g(slot)
            store(t, slot)
            if t + 1 < _NCHUNK:
                wait_ld(nxt)
                if t >= 1:
                    wait_st(nxt)
                gather(nxt)
        wait_st((_NCHUNK - 2) % 2)
        wait_st((_NCHUNK - 1) % 2)

    return k(xflat, table, off_tile)


def kernel(x, field_embedding):
    x = x.astype(jnp.int32)
    pad = jnp.zeros((_BATCH, 1), jnp.int32)
    xflat = jnp.concatenate([pad, x], axis=1).reshape(_ROWS)
    out = _sc_gather(xflat, field_embedding, jnp.asarray(_OFF_TILE))
    return out.reshape(_BATCH, _FIELDS, _EMBED_DIM)
